# baseline (speedup 1.0000x reference)
import functools

import jax
import jax.numpy as jnp
from jax import lax
from jax.experimental import pallas as pl
from jax.experimental.pallas import tpu as pltpu
from jax.experimental.pallas import tpu_sc as plsc

_L = 16
_TILE = 128
_CHT = 8
_RING = 256


@jax.jit
def kernel(node, time, embedding):
    B = node.shape[0]
    N, D = embedding.shape
    embT = embedding.T
    n_tiles = N // _TILE + 1
    tail_n = N - (n_tiles - 1) * _TILE
    tail = jnp.pad(embT[:, N - tail_n:], ((0, 0), (0, _TILE - tail_n)))
    tail = tail.reshape(D // 8, 8, _TILE)

    info = plsc.get_sparse_core_info()
    nsc = info.num_cores
    nsub = info.num_subcores
    nw = nsc * nsub
    base_t, extra = divmod(n_tiles, nw)
    n_chunks = -(-(base_t + 1) // _CHT)
    stage_words = 2 * B * D
    sh_words = stage_words // nsub

    mesh = plsc.VectorSubcoreMesh(core_axis_name="c", subcore_axis_name="s")

    @functools.partial(
        pl.kernel,
        mesh=mesh,
        compiler_params=pltpu.CompilerParams(needs_layout_passes=False),
        out_type=jax.ShapeDtypeStruct((stage_words,), jnp.float32),
        scratch_types=[
            pltpu.VMEM((2048,), jnp.int32),
            pltpu.VMEM((2080,), jnp.int32),
            pltpu.VMEM((2080,), jnp.int32),
            pltpu.VMEM((2, _CHT, D // 8, 8, _TILE), jnp.float32),
            pltpu.VMEM((_RING, D), jnp.float32),
            pltpu.SMEM((1,), jnp.int32),
            pltpu.SemaphoreType.DMA,
            pltpu.SemaphoreType.DMA,
        ],
    )
    def sc_gather(node_hbm, time_hbm, embT_hbm, tail_hbm, s_hbm,
                  idxwin, hit_idx, hit_pos, slab, ring, mcnt,
                  sem_sw, sem_st):
        c = lax.axis_index("c")
        s = lax.axis_index("s")
        w = c * nsub + s
        lo_t = w * base_t + jnp.minimum(w, extra)
        my_t = base_t + jnp.where(w < extra, 1, 0)
        hi_t = lo_t + my_t
        lo_n = lo_t * _TILE
        hi_n = hi_t * _TILE

        lanes = lax.iota(jnp.int32, _L)
        W = 2048
        scan_scope = jax.named_scope("idx_scan")
        scan_scope.__enter__()

        nh = 0
        for a, src in ((0, node_hbm), (1, time_hbm)):
            def piece(p, nh_c, a=a, src=src):
                pltpu.sync_copy(src.at[pl.ds(p * W, W)], idxwin)

                def scan(i, nh_i, a=a, p=p):
                    iv = idxwin[pl.ds(i * _L, _L)]
                    m = (iv >= lo_n) & (iv < hi_n)
                    cnt = plsc.all_reduce_population_count(m)[0]
                    pv = a * B + p * W + i * _L + lanes
                    plsc.store_compressed(
                        hit_idx.at[pl.ds(nh_i, _L)], iv, mask=m)
                    plsc.store_compressed(
                        hit_pos.at[pl.ds(nh_i, _L)], pv, mask=m)
                    return nh_i + cnt
                return lax.fori_loop(0, W // _L, scan, nh_c)
            nh = lax.fori_loop(0, B // W, piece, nh)
        scan_scope.__exit__(None, None, None)
        hit_idx[pl.ds(nh, _L)] = jnp.full((_L,), -1, jnp.int32)
        mcnt[0] = 0

        def fire(ch):
            buf = ch & 1
            t0 = lo_t + ch * _CHT
            nt = jnp.clip(hi_t - t0, 0, _CHT)

            def body(j, carry):
                ti = j >> 2
                dt = j & 3
                tile = t0 + ti

                @pl.when(tile == n_tiles - 1)
                def _():
                    pltpu.async_copy(tail_hbm.at[dt], slab.at[buf, ti, dt],
                                     sem_sw)

                @pl.when(tile < n_tiles - 1)
                def _():
                    pltpu.async_copy(
                        embT_hbm.at[pl.ds(dt * 8, 8),
                                    pl.ds(pl.multiple_of(tile * _TILE, _TILE),
                                          _TILE)],
                        slab.at[buf, ti, dt], sem_sw)
                return carry
            lax.fori_loop(0, nt * 4, body, 0)

        def drain(ch):
            buf = ch & 1
            t0 = lo_t + ch * _CHT
            nt = jnp.clip(hi_t - t0, 0, _CHT)

            def body(j, carry):
                ti = j >> 2
                dt = j & 3
                pltpu.make_async_copy(
                    tail_hbm.at[dt], slab.at[buf, ti, dt], sem_sw).wait()
                return carry
            lax.fori_loop(0, nt * 4, body, 0)

        fdt = lanes >> 3
        fsv = lanes & 7

        def process(ch):
            buf = ch & 1
            t0 = lo_t + ch * _CHT
            clo = t0 * _TILE
            chi = jnp.minimum(t0 + _CHT, hi_t) * _TILE
            nv = (nh + _L - 1) >> 4

            def rescan(k, carry):
                hv = hit_idx[pl.ds(k * _L, _L)]
                pv = hit_pos[pl.ds(k * _L, _L)]
                m2 = (hv >= clo) & (hv < chi)
                im = jnp.where(m2, 1, 0)
                any_hit = plsc.all_reduce_population_count(m2)[0]

                @pl.when(any_hit > 0)
                def _():
                    for r in range(_L):
                        @pl.when(im[r] == 1)
                        def _():
                            idx = hv[r]
                            pos = pv[r]
                            ti = (idx >> 7) - t0
                            lane = idx & (_TILE - 1)
                            bufv = jnp.full((_L,), buf, jnp.int32)
                            tiv = jnp.full((_L,), ti, jnp.int32)
                            lv = jnp.full((_L,), lane, jnp.int32)
                            v0 = plsc.load_gather(
                                slab, [bufv, tiv, fdt, fsv, lv])
                            v1 = plsc.load_gather(
                                slab, [bufv, tiv, fdt + 2, fsv, lv])
                            m = mcnt[0]
                            slot = m & (_RING - 1)

                            @pl.when(m >= _RING)
                            def _():
                                pltpu.make_async_copy(
                                    tail_hbm.at[0, 0, pl.ds(0, D)],
                                    ring.at[slot], sem_st).wait()
                            ring[slot, pl.ds(0, _L)] = v0
                            ring[slot, pl.ds(_L, _L)] = v1
                            pltpu.async_copy(
                                ring.at[slot],
                                s_hbm.at[pl.ds(pos * D, D)], sem_st)
                            mcnt[0] = m + 1
                return carry
            lax.fori_loop(0, nv, rescan, 0)

        fire(0)

        sweep_scope = jax.named_scope("sweep_extract")
        sweep_scope.__enter__()

        def chunk_loop(ch, carry):
            @pl.when(ch + 1 < n_chunks)
            def _():
                fire(ch + 1)
            drain(ch)
            process(ch)
            return carry
        lax.fori_loop(0, n_chunks, chunk_loop, 0)
        sweep_scope.__exit__(None, None, None)

        mfin = jnp.minimum(mcnt[0], _RING)

        def fdrain(i, carry):
            pltpu.make_async_copy(
                tail_hbm.at[0, 0, pl.ds(0, D)], ring.at[0], sem_st).wait()
            return carry
        lax.fori_loop(0, mfin, fdrain, 0)


    node_i = node.astype(jnp.int32)
    time_i = time.astype(jnp.int32)
    sarr = sc_gather(node_i, time_i, embT, tail)

    rows = stage_words // _TILE
    half = rows // 2
    sr = sarr.reshape(rows, _TILE)
    blk = 1024
    grid = half // blk

    def dot_kernel(sn, st, o):
        p = sn[...] * st[...]
        seg = jax.lax.broadcasted_iota(jnp.int32, (_TILE, _TILE // D), 0) // D
        col = jax.lax.broadcasted_iota(jnp.int32, (_TILE, _TILE // D), 1)
        m = jnp.where(seg == col, 1.0, 0.0).astype(jnp.float32)
        o[...] = jax.lax.dot_general(
            p, m, (((1,), (0,)), ((), ())),
            preferred_element_type=jnp.float32)

    out4 = pl.pallas_call(
        dot_kernel,
        grid=(grid,),
        in_specs=[
            pl.BlockSpec((blk, _TILE), lambda i: (i, 0)),
            pl.BlockSpec((blk, _TILE), lambda i: (i + grid, 0)),
        ],
        out_specs=pl.BlockSpec((blk, _TILE // D), lambda i: (i, 0)),
        out_shape=jax.ShapeDtypeStruct((half, _TILE // D), jnp.float32),
    )(sr, sr)

    return out4.reshape(B)


# speedup vs baseline: 3.8457x; 3.8457x over previous
import functools

import jax
import jax.numpy as jnp
from jax import lax
from jax.experimental import pallas as pl
from jax.experimental.pallas import tpu as pltpu
from jax.experimental.pallas import tpu_sc as plsc

_L = 16
_TILE = 128
_CHT = 8
_RING = 256
_WIN = 2048
_WL = 176


@jax.jit
def kernel(node, time, embedding):
    B = node.shape[0]
    N, D = embedding.shape
    embT = embedding.T
    n_tiles = N // _TILE + 1
    tail_n = N - (n_tiles - 1) * _TILE
    tail = jnp.pad(embT[:, N - tail_n:], ((0, 0), (0, _TILE - tail_n)))

    info = plsc.get_sparse_core_info()
    nsub = info.num_subcores
    nw = info.num_cores * nsub
    base_t, extra = divmod(n_tiles, nw)
    n_chunks = -(-(base_t + 1) // _CHT)
    stage_words = 2 * B * D

    mesh = plsc.VectorSubcoreMesh(core_axis_name="c", subcore_axis_name="s")

    @functools.partial(
        pl.kernel,
        mesh=mesh,
        compiler_params=pltpu.CompilerParams(needs_layout_passes=False),
        out_type=jax.ShapeDtypeStruct((stage_words,), jnp.float32),
        scratch_types=[
            pltpu.VMEM((2, _WIN), jnp.int32),
            pltpu.VMEM((2080,), jnp.int32),
            pltpu.VMEM((2080,), jnp.int32),
            pltpu.VMEM((_WL,), jnp.int32),
            pltpu.VMEM((_WL,), jnp.int32),
            pltpu.VMEM((2, _CHT, D, _TILE), jnp.float32),
            pltpu.VMEM((_RING, D), jnp.float32),
            pltpu.SemaphoreType.DMA,
            pltpu.SemaphoreType.DMA,
            pltpu.SemaphoreType.DMA,
        ],
    )
    def sc_gather(node_hbm, time_hbm, embT_hbm, tail_hbm, s_hbm,
                  idxwin, hit_idx, hit_pos, wl_idx, wl_pos, slab, ring,
                  sem_sw, sem_st, sem_ix):
        c = lax.axis_index("c")
        s = lax.axis_index("s")
        w = c * nsub + s
        lo_t = w * base_t + jnp.minimum(w, extra)
        my_t = base_t + jnp.where(w < extra, 1, 0)
        hi_t = lo_t + my_t
        lo_n = lo_t * _TILE
        hi_n = hi_t * _TILE

        lanes = lax.iota(jnp.int32, _L)
        srcs = (node_hbm, time_hbm)
        n_pieces = B // _WIN

        nh = 0
        for a in range(2):
            def piece(p, nh_c, a=a):
                buf = p & 1
                pltpu.sync_copy(srcs[a].at[pl.ds(p * _WIN, _WIN)],
                                idxwin.at[buf])

                def scan(i, nh_i, a=a):
                    iv = idxwin[buf, pl.ds(i * _L, _L)]
                    m = (iv >= lo_n) & (iv < hi_n)
                    cnt = plsc.all_reduce_population_count(m)[0]
                    pv = a * B + p * _WIN + i * _L + lanes
                    plsc.store_compressed(
                        hit_idx.at[pl.ds(nh_i, _L)], iv, mask=m)
                    plsc.store_compressed(
                        hit_pos.at[pl.ds(nh_i, _L)], pv, mask=m)
                    return nh_i + cnt
                return lax.fori_loop(0, _WIN // _L, scan, nh_c)
            nh = lax.fori_loop(0, n_pieces, piece, nh)
        hit_idx[pl.ds(nh, _L)] = jnp.full((_L,), -1, jnp.int32)

        last_full = n_tiles - 1

        def fire(ch):
            buf = ch & 1
            t0 = lo_t + ch * _CHT
            nt = jnp.clip(jnp.minimum(hi_t, last_full) - t0, 0, _CHT)

            def body(ti, carry):
                tile = t0 + ti
                pltpu.async_copy(
                    embT_hbm.at[:, pl.ds(pl.multiple_of(tile * _TILE, _TILE),
                                         _TILE)],
                    slab.at[buf, ti], sem_sw)
                return carry
            lax.fori_loop(0, nt, body, 0)
            @pl.when((t0 <= last_full) & (last_full < t0 + _CHT)
                     & (hi_t > last_full))
            def _():
                pltpu.async_copy(tail_hbm, slab.at[buf, last_full - t0],
                                 sem_sw)

        def drain(ch):
            buf = ch & 1
            t0 = lo_t + ch * _CHT
            nt = jnp.clip(jnp.minimum(hi_t, last_full) - t0, 0, _CHT)
            nt = nt + jnp.where((t0 <= last_full) & (last_full < t0 + _CHT)
                                & (hi_t > last_full), 1, 0)

            def body(ti, carry):
                pltpu.make_async_copy(
                    embT_hbm.at[:, pl.ds(0, _TILE)], slab.at[buf, ti],
                    sem_sw).wait()
                return carry
            lax.fori_loop(0, nt, body, 0)

        fire(0)

        def chunk_loop(ch, m_c):
            buf = ch & 1
            t0 = lo_t + ch * _CHT

            @pl.when(ch + 1 < n_chunks)
            def _():
                fire(ch + 1)
            drain(ch)

            clo = t0 * _TILE
            chi = jnp.minimum(t0 + _CHT, hi_t) * _TILE
            nv = (nh + _L - 1) >> 4

            def rescan(k, nc):
                hv = hit_idx[pl.ds(k * _L, _L)]
                pv = hit_pos[pl.ds(k * _L, _L)]
                m2 = (hv >= clo) & (hv < chi)
                cnt = plsc.all_reduce_population_count(m2)[0]
                plsc.store_compressed(wl_idx.at[pl.ds(nc, _L)], hv, mask=m2)
                plsc.store_compressed(wl_pos.at[pl.ds(nc, _L)], pv, mask=m2)
                return nc + cnt
            nc = lax.fori_loop(0, nv, rescan, 0)

            bufv = jnp.full((_L,), buf, jnp.int32)
            f0 = lanes
            f1 = lanes + _L

            def extract(e, m_e):
                idx = wl_idx[pl.ds(e, _L)][0]
                pos = wl_pos[pl.ds(e, _L)][0]
                tiv = jnp.full((_L,), (idx >> 7) - t0, jnp.int32)
                lv = jnp.full((_L,), idx & (_TILE - 1), jnp.int32)
                v0 = plsc.load_gather(slab, [bufv, tiv, f0, lv])
                v1 = plsc.load_gather(slab, [bufv, tiv, f1, lv])
                slot = m_e & (_RING - 1)

                @pl.when(m_e >= _RING)
                def _():
                    pltpu.make_async_copy(
                        tail_hbm.at[0, pl.ds(0, D)], ring.at[slot],
                        sem_st).wait()
                ring[slot, pl.ds(0, _L)] = v0
                ring[slot, pl.ds(_L, _L)] = v1
                pltpu.async_copy(ring.at[slot],
                                 s_hbm.at[pl.ds(pos * D, D)], sem_st)
                return m_e + 1
            return lax.fori_loop(0, nc, extract, m_c)

        m_fin = lax.fori_loop(0, n_chunks, chunk_loop, 0)

        def fdrain(i, carry):
            pltpu.make_async_copy(
                tail_hbm.at[0, pl.ds(0, D)], ring.at[0], sem_st).wait()
            return carry
        lax.fori_loop(0, jnp.minimum(m_fin, _RING), fdrain, 0)

    node_i = node.astype(jnp.int32)
    time_i = time.astype(jnp.int32)
    sarr = sc_gather(node_i, time_i, embT, tail)

    rows = stage_words // _TILE
    half = rows // 2
    sr = sarr.reshape(rows, _TILE)
    blk = 1024
    grid = half // blk

    def dot_kernel(sn, st, o):
        p = sn[...] * st[...]
        seg = jax.lax.broadcasted_iota(jnp.int32, (_TILE, _TILE // D), 0) // D
        col = jax.lax.broadcasted_iota(jnp.int32, (_TILE, _TILE // D), 1)
        m = jnp.where(seg == col, 1.0, 0.0).astype(jnp.float32)
        o[...] = jax.lax.dot_general(
            p, m, (((1,), (0,)), ((), ())),
            preferred_element_type=jnp.float32)

    out4 = pl.pallas_call(
        dot_kernel,
        grid=(grid,),
        in_specs=[
            pl.BlockSpec((blk, _TILE), lambda i: (i, 0)),
            pl.BlockSpec((blk, _TILE), lambda i: (i + grid, 0)),
        ],
        out_specs=pl.BlockSpec((blk, _TILE // D), lambda i: (i, 0)),
        out_shape=jax.ShapeDtypeStruct((half, _TILE // D), jnp.float32),
    )(sr, sr)

    return out4.reshape(B)


# speedup vs baseline: 3.9040x; 1.0151x over previous
import functools

import jax
import jax.numpy as jnp
from jax import lax
from jax.experimental import pallas as pl
from jax.experimental.pallas import tpu as pltpu
from jax.experimental.pallas import tpu_sc as plsc

_L = 16
_TILE = 128
_CHT = 8
_RING = 256
_WIN = 2048
_WL = 176


@jax.jit
def kernel(node, time, embedding):
    B = node.shape[0]
    N, D = embedding.shape
    embT = embedding.T
    n_tiles = N // _TILE + 1
    tail_n = N - (n_tiles - 1) * _TILE
    tail = jnp.pad(embT[:, N - tail_n:], ((0, 0), (0, _TILE - tail_n)))

    info = plsc.get_sparse_core_info()
    nsub = info.num_subcores
    nw = info.num_cores * nsub
    base_t, extra = divmod(n_tiles, nw)
    n_chunks = -(-(base_t + 1) // _CHT)
    stage_words = 2 * B * D

    mesh = plsc.VectorSubcoreMesh(core_axis_name="c", subcore_axis_name="s")

    @functools.partial(
        pl.kernel,
        mesh=mesh,
        compiler_params=pltpu.CompilerParams(needs_layout_passes=False),
        out_type=jax.ShapeDtypeStruct((stage_words,), jnp.float32),
        scratch_types=[
            pltpu.VMEM((2, _WIN), jnp.int32),
            pltpu.VMEM((2080,), jnp.int32),
            pltpu.VMEM((2080,), jnp.int32),
            pltpu.VMEM((_WL,), jnp.int32),
            pltpu.VMEM((_WL,), jnp.int32),
            pltpu.VMEM((2, D, _CHT * _TILE), jnp.float32),
            pltpu.VMEM((_RING, D), jnp.float32),
            pltpu.SemaphoreType.DMA,
            pltpu.SemaphoreType.DMA,
            pltpu.SemaphoreType.DMA,
        ],
    )
    def sc_gather(node_hbm, time_hbm, embT_hbm, tail_hbm, s_hbm,
                  idxwin, hit_idx, hit_pos, wl_idx, wl_pos, slab, ring,
                  sem_sw, sem_st, sem_ix):
        c = lax.axis_index("c")
        s = lax.axis_index("s")
        w = c * nsub + s
        lo_t = w * base_t + jnp.minimum(w, extra)
        my_t = base_t + jnp.where(w < extra, 1, 0)
        hi_t = lo_t + my_t
        lo_n = lo_t * _TILE
        hi_n = hi_t * _TILE

        lanes = lax.iota(jnp.int32, _L)
        srcs = (node_hbm, time_hbm)
        n_pieces = B // _WIN

        nh = 0
        for a in range(2):
            def piece(p, nh_c, a=a):
                buf = p & 1
                pltpu.sync_copy(srcs[a].at[pl.ds(p * _WIN, _WIN)],
                                idxwin.at[buf])

                def scan(i, nh_i, a=a):
                    for u in range(4):
                        iv = idxwin[buf, pl.ds((i * 4 + u) * _L, _L)]
                        m = (iv >= lo_n) & (iv < hi_n)
                        cnt = plsc.all_reduce_population_count(m)[0]
                        pv = a * B + p * _WIN + (i * 4 + u) * _L + lanes
                        plsc.store_compressed(
                            hit_idx.at[pl.ds(nh_i, _L)], iv, mask=m)
                        plsc.store_compressed(
                            hit_pos.at[pl.ds(nh_i, _L)], pv, mask=m)
                        nh_i = nh_i + cnt
                    return nh_i
                return lax.fori_loop(0, _WIN // (_L * 4), scan, nh_c)
            nh = lax.fori_loop(0, n_pieces, piece, nh)
        hit_idx[pl.ds(nh, _L)] = jnp.full((_L,), -1, jnp.int32)

        last_full = n_tiles - 1

        def fire(ch):
            buf = ch & 1
            t0 = lo_t + ch * _CHT
            full_w = _CHT * _TILE

            @pl.when(t0 + _CHT <= jnp.minimum(hi_t, last_full))
            def _():
                pltpu.async_copy(
                    embT_hbm.at[:, pl.ds(
                        pl.multiple_of(t0 * _TILE, _TILE), full_w)],
                    slab.at[buf], sem_sw)

            @pl.when(t0 + _CHT > jnp.minimum(hi_t, last_full))
            def _():
                nt = jnp.clip(jnp.minimum(hi_t, last_full) - t0, 0, _CHT)

                def body(ti, carry):
                    pltpu.async_copy(
                        embT_hbm.at[:, pl.ds(
                            pl.multiple_of((t0 + ti) * _TILE, _TILE), _TILE)],
                        slab.at[buf, :, pl.ds(ti * _TILE, _TILE)], sem_sw)
                    return carry
                lax.fori_loop(0, nt, body, 0)
                @pl.when((t0 <= last_full) & (last_full < t0 + _CHT)
                         & (hi_t > last_full))
                def _():
                    pltpu.async_copy(
                        tail_hbm,
                        slab.at[buf, :, pl.ds((last_full - t0) * _TILE,
                                              _TILE)], sem_sw)

        def drain(ch):
            buf = ch & 1
            t0 = lo_t + ch * _CHT

            @pl.when(t0 + _CHT <= jnp.minimum(hi_t, last_full))
            def _():
                pltpu.make_async_copy(
                    embT_hbm.at[:, pl.ds(0, _CHT * _TILE)], slab.at[buf],
                    sem_sw).wait()

            @pl.when(t0 + _CHT > jnp.minimum(hi_t, last_full))
            def _():
                nt = jnp.clip(jnp.minimum(hi_t, last_full) - t0, 0, _CHT)
                nt = nt + jnp.where(
                    (t0 <= last_full) & (last_full < t0 + _CHT)
                    & (hi_t > last_full), 1, 0)

                def body(ti, carry):
                    pltpu.make_async_copy(
                        embT_hbm.at[:, pl.ds(0, _TILE)],
                        slab.at[buf, :, pl.ds(0, _TILE)], sem_sw).wait()
                    return carry
                lax.fori_loop(0, nt, body, 0)

        fire(0)
        fire(1)

        def chunk_loop(ch, m_c):
            buf = ch & 1
            t0 = lo_t + ch * _CHT

            drain(ch)

            clo = t0 * _TILE
            chi = jnp.minimum(t0 + _CHT, hi_t) * _TILE
            nv = (nh + _L - 1) >> 4

            def rescan(k, nc):
                hv = hit_idx[pl.ds(k * _L, _L)]
                pv = hit_pos[pl.ds(k * _L, _L)]
                m2 = (hv >= clo) & (hv < chi)
                cnt = plsc.all_reduce_population_count(m2)[0]
                plsc.store_compressed(wl_idx.at[pl.ds(nc, _L)], hv, mask=m2)
                plsc.store_compressed(wl_pos.at[pl.ds(nc, _L)], pv, mask=m2)
                return nc + cnt
            nc = lax.fori_loop(0, nv, rescan, 0)

            bufv = jnp.full((_L,), buf, jnp.int32)
            f0 = lanes
            f1 = lanes + _L
            base_n = t0 * _TILE

            def extract(e, m_e):
                idx = wl_idx[pl.ds(e, _L)][0]
                pos = wl_pos[pl.ds(e, _L)][0]
                lv = jnp.full((_L,), idx - base_n, jnp.int32)
                v0 = plsc.load_gather(slab, [bufv, f0, lv])
                v1 = plsc.load_gather(slab, [bufv, f1, lv])
                slot = m_e & (_RING - 1)

                @pl.when(m_e >= _RING)
                def _():
                    pltpu.make_async_copy(
                        tail_hbm.at[0, pl.ds(0, D)], ring.at[slot],
                        sem_st).wait()
                ring[slot, pl.ds(0, _L)] = v0
                ring[slot, pl.ds(_L, _L)] = v1
                pltpu.async_copy(ring.at[slot],
                                 s_hbm.at[pl.ds(pos * D, D)], sem_st)
                return m_e + 1
            m_new = lax.fori_loop(0, nc, extract, m_c)

            @pl.when(ch + 2 < n_chunks)
            def _():
                fire(ch + 2)
            return m_new

        m_fin = lax.fori_loop(0, n_chunks, chunk_loop, 0)

        def fdrain(i, carry):
            pltpu.make_async_copy(
                tail_hbm.at[0, pl.ds(0, D)], ring.at[0], sem_st).wait()
            return carry
        lax.fori_loop(0, jnp.minimum(m_fin, _RING), fdrain, 0)

    node_i = node.astype(jnp.int32)
    time_i = time.astype(jnp.int32)
    sarr = sc_gather(node_i, time_i, embT, tail)

    rows = stage_words // _TILE
    half = rows // 2
    sr = sarr.reshape(rows, _TILE)
    blk = 1024
    grid = half // blk

    def dot_kernel(sn, st, o):
        p = sn[...] * st[...]
        seg = jax.lax.broadcasted_iota(jnp.int32, (_TILE, _TILE // D), 0) // D
        col = jax.lax.broadcasted_iota(jnp.int32, (_TILE, _TILE // D), 1)
        m = jnp.where(seg == col, 1.0, 0.0).astype(jnp.float32)
        o[...] = jax.lax.dot_general(
            p, m, (((1,), (0,)), ((), ())),
            preferred_element_type=jnp.float32)

    out4 = pl.pallas_call(
        dot_kernel,
        grid=(grid,),
        in_specs=[
            pl.BlockSpec((blk, _TILE), lambda i: (i, 0)),
            pl.BlockSpec((blk, _TILE), lambda i: (i + grid, 0)),
        ],
        out_specs=pl.BlockSpec((blk, _TILE // D), lambda i: (i, 0)),
        out_shape=jax.ShapeDtypeStruct((half, _TILE // D), jnp.float32),
    )(sr, sr)

    return out4.reshape(B)
